# baseline (speedup 1.0000x reference)
import functools

import jax
import jax.numpy as jnp
from jax import lax
from jax.experimental import pallas as pl
from jax.experimental.pallas import tpu as pltpu
from jax.experimental.pallas import tpu_sc as plsc

F = 16
HID = 128
NC, NS = 2, 16
NW = NC * NS
GCHUNK = 40
GK = 5
GROW = GCHUNK * GK


def _sc_gather_body(table, row3d, out, idx_v, buf_a, buf_b,
                    gsem_a, gsem_b, wsem_a, wsem_b):
    c = lax.axis_index("c")
    s = lax.axis_index("s")
    wid = s * NC + c
    ept = out.shape[0] // NW
    ngroups = ept // GROW
    base_out = wid * ept

    pltpu.sync_copy(row3d.at[wid], idx_v)

    def fire_gathers(g, buf, gsem):
        for b in range(GK):
            pltpu.async_copy(table.at[idx_v.at[g * GK + b]],
                             buf.at[pl.ds(b * GCHUNK, GCHUNK)], gsem)

    def wait_gathers(buf, gsem):
        pltpu.make_async_copy(table.at[pl.ds(0, GROW)], buf, gsem).wait()

    def fire_write(g, buf, wsem):
        pltpu.async_copy(buf, out.at[pl.ds(base_out + g * GROW, GROW)], wsem)

    def wait_write(buf, wsem):
        pltpu.make_async_copy(buf, out.at[pl.ds(base_out, GROW)], wsem).wait()

    fire_gathers(0, buf_a, gsem_a)
    fire_gathers(1, buf_b, gsem_b)

    def body(p, carry):
        g0 = 2 * p
        wait_gathers(buf_a, gsem_a)
        fire_write(g0, buf_a, wsem_a)
        wait_gathers(buf_b, gsem_b)
        fire_write(g0 + 1, buf_b, wsem_b)

        @pl.when(p < ngroups // 2 - 1)
        def _refill():
            wait_write(buf_a, wsem_a)
            fire_gathers(g0 + 2, buf_a, gsem_a)
            wait_write(buf_b, wsem_b)
            fire_gathers(g0 + 3, buf_b, gsem_b)

        return carry

    lax.fori_loop(0, ngroups // 2, body, 0)
    wait_write(buf_a, wsem_a)
    wait_write(buf_b, wsem_b)


def _sc_gather(table, row):
    E = row.shape[0]
    width = table.shape[1]
    ept = E // NW
    mesh = plsc.VectorSubcoreMesh(core_axis_name="c", subcore_axis_name="s")
    return pl.kernel(
        _sc_gather_body,
        out_type=jax.ShapeDtypeStruct((E, width), jnp.float32),
        mesh=mesh,
        compiler_params=pltpu.CompilerParams(use_tc_tiling_on_sc=False),
        scratch_types=[
            pltpu.VMEM((ept // GCHUNK, GCHUNK), jnp.int32),
            pltpu.VMEM((GROW, width), jnp.float32),
            pltpu.VMEM((GROW, width), jnp.float32),
            pltpu.SemaphoreType.DMA,
            pltpu.SemaphoreType.DMA,
            pltpu.SemaphoreType.DMA,
            pltpu.SemaphoreType.DMA,
        ],
    )(table, row.reshape(NW, ept // GCHUNK, GCHUNK))


SK = 5
SROW = GCHUNK * SK


HF = F // 2
QF = F // 4


SCH = 3128
WCH = 368


def _make_scatter_body(with_cnt):
  def _sc_scatter_body(v_lo, v_hi, col4d, zeros2d, zeros1d, ones_h,
                       sum_lo, sum_hi, cnt_o,
                       idx_v, vbuf_a, vbuf_b, ones_v, wbuf, cbuf,
                       acc, cnt_acc, gsem_a, gsem_b):
    c = lax.axis_index("c")
    s = lax.axis_index("s")
    E = v_lo.shape[0]
    N = sum_lo.shape[0]
    ept = E // NS
    ngroups = ept // SROW
    base_e = s * ept
    node_base = s * 2 * SCH

    pltpu.sync_copy(ones_h, ones_v)
    pltpu.sync_copy(zeros2d, wbuf)
    pltpu.sync_copy(zeros1d, cbuf)

    def zero_acc(acc, with_cnt):
        for k in range(17):
            pltpu.sync_copy(wbuf, acc.at[pl.ds(node_base + k * WCH, WCH)])
            if with_cnt:
                pltpu.sync_copy(cbuf,
                                cnt_acc.at[pl.ds(node_base + k * WCH, WCH)])

    def writeback(acc, out_lo, out_hi, with_cnt):
        for k in range(17):
            cb = node_base + k * WCH
            pltpu.sync_copy(acc.at[pl.ds(cb, WCH)], wbuf)

            @pl.when(c == 0)
            def _lo():
                pltpu.sync_copy(wbuf, out_lo.at[pl.ds(cb, WCH)])

            @pl.when(c == 1)
            def _hi():
                pltpu.sync_copy(wbuf, out_hi.at[pl.ds(cb, WCH)])

            if with_cnt:
                @pl.when(c == 1)
                def _cnt():
                    pltpu.sync_copy(cnt_acc.at[pl.ds(cb, WCH)], cbuf)
                    pltpu.sync_copy(cbuf, cnt_o.at[pl.ds(cb, WCH)])

    def run_phase(acc, v_lo, v_hi, h, with_cnt):
        pltpu.sync_copy(col4d.at[s, h], idx_v)
        goff = h * (ngroups // 2)
        def fire_load(g, buf, gsem):
            @pl.when(c == 0)
            def _lo():
                pltpu.async_copy(
                    v_lo.at[pl.ds(base_e + (goff + g) * SROW, SROW)],
                    buf, gsem)

            @pl.when(c == 1)
            def _hi():
                pltpu.async_copy(
                    v_hi.at[pl.ds(base_e + (goff + g) * SROW, SROW)],
                    buf, gsem)

        def wait_load(buf, gsem):
            pltpu.make_async_copy(v_lo.at[pl.ds(0, SROW)], buf, gsem).wait()

        def scatter_group(g, buf):
            for b in range(SK):
                idx = idx_v.at[g * SK + b]
                pltpu.sync_copy(buf.at[pl.ds(b * GCHUNK, GCHUNK)],
                                acc.at[idx], add=True)
                if with_cnt:
                    @pl.when(c == 1)
                    def _cnt():
                        pltpu.sync_copy(ones_v.at[pl.ds(0, GCHUNK)],
                                        cnt_acc.at[idx], add=True)

        fire_load(0, vbuf_a, gsem_a)
        fire_load(1, vbuf_b, gsem_b)

        def body(p, carry):
            g0 = 2 * p
            wait_load(vbuf_a, gsem_a)
            scatter_group(g0, vbuf_a)

            @pl.when(p < ngroups // 4 - 1)
            def _next_a():
                fire_load(g0 + 2, vbuf_a, gsem_a)

            wait_load(vbuf_b, gsem_b)
            scatter_group(g0 + 1, vbuf_b)

            @pl.when(p < ngroups // 4 - 1)
            def _next_b():
                fire_load(g0 + 3, vbuf_b, gsem_b)

            return carry

        lax.fori_loop(0, ngroups // 4, body, 0)

    zero_acc(acc, with_cnt=with_cnt)
    plsc.subcore_barrier()
    run_phase(acc, v_lo, v_hi, 0, with_cnt=with_cnt)
    run_phase(acc, v_lo, v_hi, 1, with_cnt=with_cnt)
    plsc.subcore_barrier()
    writeback(acc, sum_lo, sum_hi, with_cnt=with_cnt)

  return _sc_scatter_body


def _sc_scatter(quarters, col, n):
    E = quarters[0].shape[0]
    ept = E // NS
    npad = NS * 2 * SCH
    assert npad >= n
    mesh = plsc.VectorSubcoreMesh(core_axis_name="c", subcore_axis_name="s")
    shp = jax.ShapeDtypeStruct
    col4d = col.reshape(NS, 2, ept // 2 // GCHUNK, GCHUNK)
    z2 = jnp.zeros((WCH, QF), jnp.float32)
    z1 = jnp.zeros((WCH,), jnp.float32)
    on = jnp.ones((48,), jnp.float32)

    sums = []
    for k in range(4):
        with_cnt = False
        res = pl.kernel(
            _make_scatter_body(with_cnt),
            out_type=[shp((npad, QF), jnp.float32)] * 2
                     + [shp((npad,), jnp.float32)],
            mesh=mesh,
            compiler_params=pltpu.CompilerParams(use_tc_tiling_on_sc=False),
            scratch_types=[
                pltpu.VMEM((ept // 2 // GCHUNK, GCHUNK), jnp.int32),
                pltpu.VMEM((SROW, QF), jnp.float32),
                pltpu.VMEM((SROW, QF), jnp.float32),
                pltpu.VMEM((48,), jnp.float32),
                pltpu.VMEM((WCH, QF), jnp.float32),
                pltpu.VMEM((WCH,), jnp.float32),
                pltpu.VMEM_SHARED((npad, QF), jnp.float32),
                pltpu.VMEM_SHARED((npad,) if with_cnt else (8,), jnp.float32),
                pltpu.SemaphoreType.DMA,
                pltpu.SemaphoreType.DMA,
            ],
            name=f"sc_scatter_{k}",
        )(quarters[2 * k], quarters[2 * k + 1], col4d, z2, z1, on)
        sums.extend(r[:n] for r in res[:2])
    cnt0, cnt1 = _sc_count(col4d, z2, n)
    return sums + [cnt0, cnt1]


def _sc_count_body(col4d, zeros2d, ones2_h, cnt0, cnt1,
                   idx_v, ones_v, wbuf, acc):
    c = lax.axis_index("c")
    s = lax.axis_index("s")
    nchunks = idx_v.shape[0]
    node_base = s * 2 * SCH

    pltpu.sync_copy(col4d.at[s, c], idx_v)
    pltpu.sync_copy(ones2_h, ones_v)
    pltpu.sync_copy(zeros2d, wbuf)

    for k in range(17):
        pltpu.sync_copy(wbuf, acc.at[pl.ds(node_base + k * WCH, WCH)])
    plsc.subcore_barrier()

    def body(j, carry):
        pltpu.sync_copy(ones_v.at[pl.ds(0, GCHUNK)],
                        acc.at[idx_v.at[j]], add=True)
        return carry

    lax.fori_loop(0, nchunks, body, 0)
    plsc.subcore_barrier()

    for k in range(17):
        cb = node_base + k * WCH
        pltpu.sync_copy(acc.at[pl.ds(cb, WCH)], wbuf)

        @pl.when(c == 0)
        def _c0():
            pltpu.sync_copy(wbuf, cnt0.at[pl.ds(cb, WCH)])

        @pl.when(c == 1)
        def _c1():
            pltpu.sync_copy(wbuf, cnt1.at[pl.ds(cb, WCH)])


def _sc_count(col4d, z2, n):
    npad = NS * 2 * SCH
    nchunks = col4d.shape[2]
    mesh = plsc.VectorSubcoreMesh(core_axis_name="c", subcore_axis_name="s")
    shp = jax.ShapeDtypeStruct
    res = pl.kernel(
        _sc_count_body,
        out_type=[shp((npad, QF), jnp.float32)] * 2,
        mesh=mesh,
        compiler_params=pltpu.CompilerParams(use_tc_tiling_on_sc=False),
        scratch_types=[
            pltpu.VMEM((nchunks, GCHUNK), jnp.int32),
            pltpu.VMEM((48, QF), jnp.float32),
            pltpu.VMEM((WCH, QF), jnp.float32),
            pltpu.VMEM_SHARED((npad, QF), jnp.float32),
        ],
        name="sc_count",
    )(col4d, z2, jnp.ones((48, QF), jnp.float32))
    return res[0][:n, :1], res[1][:n, :1]


def _edge_stats_kernel(pag, ea, w1p, b1p, w1a, b1a,
                       sum_p, sq_p, sum_a, sq_a):
    @pl.when(pl.program_id(0) == 0)
    def _init():
        sum_p[...] = jnp.zeros_like(sum_p)
        sq_p[...] = jnp.zeros_like(sq_p)
        sum_a[...] = jnp.zeros_like(sum_a)
        sq_a[...] = jnp.zeros_like(sq_a)

    g = pag[...]
    e = ea[...]
    xp = jnp.concatenate([g[:, :F], e], axis=1)
    xa = jnp.concatenate([g[:, F:], e], axis=1)
    hp = jnp.maximum(jnp.dot(xp, w1p[...],
                             preferred_element_type=jnp.float32) + b1p[...], 0.0)
    ha = jnp.maximum(jnp.dot(xa, w1a[...],
                             preferred_element_type=jnp.float32) + b1a[...], 0.0)
    sum_p[...] += jnp.sum(hp, axis=0, keepdims=True)
    sq_p[...] += jnp.sum(hp * hp, axis=0, keepdims=True)
    sum_a[...] += jnp.sum(ha, axis=0, keepdims=True)
    sq_a[...] += jnp.sum(ha * ha, axis=0, keepdims=True)


def _edge_mlp_kernel(nrows, pag, ea,
                     w1p, b1p, gp, btp, w2p, b2p, w3p, b3p,
                     w1a, b1a, ga, bta, w2a, b2a, w3a, b3a,
                     sum_p, sq_p, sum_a, sq_a, *outs):
    g = pag[...]
    e = ea[...]

    def branch(xcols, w1, b1, gamma, beta, w2, b2, w3, b3, s, sq):
        x = jnp.concatenate([xcols, e], axis=1)
        h = jnp.maximum(jnp.dot(x, w1[...],
                                preferred_element_type=jnp.float32) + b1[...], 0.0)
        mu = s[...] / nrows
        var = sq[...] / nrows - mu * mu
        scale = gamma[...] * lax.rsqrt(var + 1e-5)
        hn = (h - mu) * scale + beta[...]
        h2 = jnp.maximum(jnp.dot(hn, w2[...],
                                 preferred_element_type=jnp.float32) + b2[...], 0.0)
        return jnp.dot(h2, w3[...], preferred_element_type=jnp.float32) + b3[...]

    op = branch(g[:, :F], w1p, b1p, gp, btp, w2p, b2p, w3p, b3p,
                sum_p, sq_p)
    oa = branch(g[:, F:], w1a, b1a, ga, bta, w2a, b2a, w3a, b3a,
                sum_a, sq_a)
    outs[0][...] = op
    outs[1][...] = oa


def _run_edge_mlps(pa_g, edge_attr, p1, a1):
    E = pa_g.shape[0]
    BE = 3200
    grid = (E // BE,)
    vec = lambda name, p: p[name].reshape(1, -1)
    full = lambda a: pl.BlockSpec(a.shape, lambda i: (0,) * a.ndim)
    blk = lambda w: pl.BlockSpec((BE, w), lambda i: (i, 0))
    stats_spec = [pl.BlockSpec((1, HID), lambda i: (0, 0))] * 4

    wargs1 = (p1['W1'], vec('b1', p1), a1['W1'], vec('b1', a1))
    stats = pl.pallas_call(
        _edge_stats_kernel,
        grid=grid,
        in_specs=[blk(2 * F), blk(F)] + [full(w) for w in wargs1],
        out_specs=stats_spec,
        out_shape=[jax.ShapeDtypeStruct((1, HID), jnp.float32)] * 4,
        compiler_params=pltpu.CompilerParams(
            dimension_semantics=("arbitrary",)),
    )(pa_g, edge_attr, *wargs1)

    def wset(p):
        return (p['W1'], vec('b1', p), vec('gamma', p), vec('beta', p),
                p['W2'], vec('b2', p), p['W3'], vec('b3', p))

    wargs2 = wset(p1) + wset(a1)
    outs = pl.pallas_call(
        functools.partial(_edge_mlp_kernel, float(E)),
        grid=grid,
        in_specs=([blk(2 * F), blk(F)] + [full(w) for w in wargs2]
                  + stats_spec),
        out_specs=[blk(F)] * 2,
        out_shape=[jax.ShapeDtypeStruct((E, F), jnp.float32)] * 2,
        compiler_params=pltpu.CompilerParams(
            dimension_semantics=("arbitrary",)),
    )(pa_g, edge_attr, *wargs2, *stats)
    return outs


def _node_x(nodes, sums, cnts):
    c = cnts[...]
    inv = 1.0 / jnp.maximum(c[:, :1] + c[:, 1:2], 1.0)
    return jnp.concatenate([nodes[...], sums[...] * inv], axis=1)


def _node_stats_kernel(nodes, sums, cnts, w1, b1, sum_o, sq_o):
    @pl.when(pl.program_id(0) == 0)
    def _init():
        sum_o[...] = jnp.zeros_like(sum_o)
        sq_o[...] = jnp.zeros_like(sq_o)

    x = _node_x(nodes, sums, cnts)
    h = jnp.maximum(jnp.dot(x, w1[...],
                            preferred_element_type=jnp.float32) + b1[...], 0.0)
    sum_o[...] += jnp.sum(h, axis=0, keepdims=True)
    sq_o[...] += jnp.sum(h * h, axis=0, keepdims=True)


def _node_mlp_kernel(nrows, nodes, sums, cnts,
                     w1, b1, gamma, beta, w2, b2, w3, b3, s, sq, out):
    x = _node_x(nodes, sums, cnts)
    h = jnp.maximum(jnp.dot(x, w1[...],
                            preferred_element_type=jnp.float32) + b1[...], 0.0)
    mu = s[...] / nrows
    var = sq[...] / nrows - mu * mu
    scale = gamma[...] * lax.rsqrt(var + 1e-5)
    hn = (h - mu) * scale + beta[...]
    h2 = jnp.maximum(jnp.dot(hn, w2[...],
                             preferred_element_type=jnp.float32) + b2[...], 0.0)
    out[...] = (nodes[...] + jnp.dot(h2, w3[...],
                                     preferred_element_type=jnp.float32)
                + b3[...])


def _run_node_mlp(nodes, sums, cnts, p):
    N = nodes.shape[0]
    BN = 4000
    grid = (N // BN,)
    vec = lambda name: p[name].reshape(1, -1)
    full = lambda a: pl.BlockSpec(a.shape, lambda i: (0,) * a.ndim)
    blk = lambda w: pl.BlockSpec((BN, w), lambda i: (i, 0))
    stats_spec = [pl.BlockSpec((1, HID), lambda i: (0, 0))] * 2
    data_specs = [blk(F), blk(F), blk(2)]

    wargs1 = (p['W1'], vec('b1'))
    stats = pl.pallas_call(
        _node_stats_kernel,
        grid=grid,
        in_specs=data_specs + [full(w) for w in wargs1],
        out_specs=stats_spec,
        out_shape=[jax.ShapeDtypeStruct((1, HID), jnp.float32)] * 2,
        compiler_params=pltpu.CompilerParams(
            dimension_semantics=("arbitrary",)),
    )(nodes, sums, cnts, *wargs1)

    wargs2 = (p['W1'], vec('b1'), vec('gamma'), vec('beta'),
              p['W2'], vec('b2'), p['W3'], vec('b3'))
    out = pl.pallas_call(
        functools.partial(_node_mlp_kernel, float(N)),
        grid=grid,
        in_specs=data_specs + [full(w) for w in wargs2] + stats_spec,
        out_specs=blk(F),
        out_shape=jax.ShapeDtypeStruct((N, F), jnp.float32),
        compiler_params=pltpu.CompilerParams(
            dimension_semantics=("arbitrary",)),
    )(nodes, sums, cnts, *wargs2, *stats)
    return out


def kernel(pos, ang, edge_index, edge_attr, params):
    row = edge_index[0]
    col = edge_index[1]
    n = pos.shape[0]

    pa = jnp.concatenate([pos, ang], axis=1)
    pa_g = _sc_gather(pa, row)

    op_full, oa_full = _run_edge_mlps(pa_g, edge_attr,
                                      params['p1'], params['a1'])
    equarters = ([op_full[:, q * QF:(q + 1) * QF] for q in range(4)]
                 + [oa_full[:, q * QF:(q + 1) * QF] for q in range(4)])

    *sums, cnt0, cnt1 = _sc_scatter(equarters, col, n)
    cnts = jnp.concatenate([cnt0, cnt1], axis=1)
    sums_p = jnp.concatenate(sums[:4], axis=1)
    sums_a = jnp.concatenate(sums[4:], axis=1)

    u = _run_node_mlp(pos, sums_p, cnts, params['p2'])
    phi = _run_node_mlp(ang, sums_a, cnts, params['a2'])
    return (u, phi)


# speedup vs baseline: 1.1631x; 1.1631x over previous
import functools

import jax
import jax.numpy as jnp
from jax import lax
from jax.experimental import pallas as pl
from jax.experimental.pallas import tpu as pltpu
from jax.experimental.pallas import tpu_sc as plsc

F = 16
HID = 128
NC, NS = 2, 16
NW = NC * NS
GCHUNK = 40
GK = 5
GROW = GCHUNK * GK


def _sc_gather_body(table, row3d, out, idx_v, buf_a, buf_b,
                    gsem_a, gsem_b, wsem_a, wsem_b):
    c = lax.axis_index("c")
    s = lax.axis_index("s")
    wid = s * NC + c
    ept = out.shape[0] // NW
    ngroups = ept // GROW
    base_out = wid * ept

    pltpu.sync_copy(row3d.at[wid], idx_v)

    def fire_gathers(g, buf, gsem):
        for b in range(GK):
            pltpu.async_copy(table.at[idx_v.at[g * GK + b]],
                             buf.at[pl.ds(b * GCHUNK, GCHUNK)], gsem)

    def wait_gathers(buf, gsem):
        pltpu.make_async_copy(table.at[pl.ds(0, GROW)], buf, gsem).wait()

    def fire_write(g, buf, wsem):
        pltpu.async_copy(buf, out.at[pl.ds(base_out + g * GROW, GROW)], wsem)

    def wait_write(buf, wsem):
        pltpu.make_async_copy(buf, out.at[pl.ds(base_out, GROW)], wsem).wait()

    fire_gathers(0, buf_a, gsem_a)
    fire_gathers(1, buf_b, gsem_b)

    def body(p, carry):
        g0 = 2 * p
        wait_gathers(buf_a, gsem_a)
        fire_write(g0, buf_a, wsem_a)
        wait_gathers(buf_b, gsem_b)
        fire_write(g0 + 1, buf_b, wsem_b)

        @pl.when(p < ngroups // 2 - 1)
        def _refill():
            wait_write(buf_a, wsem_a)
            fire_gathers(g0 + 2, buf_a, gsem_a)
            wait_write(buf_b, wsem_b)
            fire_gathers(g0 + 3, buf_b, gsem_b)

        return carry

    lax.fori_loop(0, ngroups // 2, body, 0)
    wait_write(buf_a, wsem_a)
    wait_write(buf_b, wsem_b)


def _sc_gather(table, row):
    E = row.shape[0]
    width = table.shape[1]
    ept = E // NW
    mesh = plsc.VectorSubcoreMesh(core_axis_name="c", subcore_axis_name="s")
    return pl.kernel(
        _sc_gather_body,
        out_type=jax.ShapeDtypeStruct((E, width), jnp.float32),
        mesh=mesh,
        compiler_params=pltpu.CompilerParams(use_tc_tiling_on_sc=False),
        scratch_types=[
            pltpu.VMEM((ept // GCHUNK, GCHUNK), jnp.int32),
            pltpu.VMEM((GROW, width), jnp.float32),
            pltpu.VMEM((GROW, width), jnp.float32),
            pltpu.SemaphoreType.DMA,
            pltpu.SemaphoreType.DMA,
            pltpu.SemaphoreType.DMA,
            pltpu.SemaphoreType.DMA,
        ],
    )(table, row.reshape(NW, ept // GCHUNK, GCHUNK))


SK = 5
SROW = GCHUNK * SK


HF = F // 2
QF = F // 4


SCH = 3128
WCH = 368


def _make_scatter_body(with_cnt):
  def _sc_scatter_body(v_lo, v_hi, col4d, zeros2d, zeros1d, ones_h,
                       sum_lo, sum_hi, cnt_o,
                       idx_v, vbuf_a, vbuf_b, ones_v, wbuf, cbuf,
                       acc, cnt_acc, gsem_a, gsem_b):
    c = lax.axis_index("c")
    s = lax.axis_index("s")
    E = v_lo.shape[0]
    N = sum_lo.shape[0]
    ept = E // NS
    ngroups = ept // SROW
    base_e = s * ept
    node_base = s * 2 * SCH

    pltpu.sync_copy(ones_h, ones_v)
    pltpu.sync_copy(zeros2d, wbuf)
    pltpu.sync_copy(zeros1d, cbuf)

    def zero_acc(acc, with_cnt):
        for k in range(17):
            pltpu.sync_copy(wbuf, acc.at[pl.ds(node_base + k * WCH, WCH)])
            if with_cnt:
                pltpu.sync_copy(cbuf,
                                cnt_acc.at[pl.ds(node_base + k * WCH, WCH)])

    def writeback(acc, out_lo, out_hi, with_cnt):
        for k in range(17):
            cb = node_base + k * WCH
            pltpu.sync_copy(acc.at[pl.ds(cb, WCH)], wbuf)

            @pl.when(c == 0)
            def _lo():
                pltpu.sync_copy(wbuf, out_lo.at[pl.ds(cb, WCH)])

            @pl.when(c == 1)
            def _hi():
                pltpu.sync_copy(wbuf, out_hi.at[pl.ds(cb, WCH)])

            if with_cnt:
                @pl.when(c == 1)
                def _cnt():
                    pltpu.sync_copy(cnt_acc.at[pl.ds(cb, WCH)], cbuf)
                    pltpu.sync_copy(cbuf, cnt_o.at[pl.ds(cb, WCH)])

    def run_phase(acc, v_lo, v_hi, h, with_cnt):
        pltpu.sync_copy(col4d.at[s, h], idx_v)
        goff = h * (ngroups // 2)
        def fire_load(g, buf, gsem):
            @pl.when(c == 0)
            def _lo():
                pltpu.async_copy(
                    v_lo.at[pl.ds(base_e + (goff + g) * SROW, SROW)],
                    buf, gsem)

            @pl.when(c == 1)
            def _hi():
                pltpu.async_copy(
                    v_hi.at[pl.ds(base_e + (goff + g) * SROW, SROW)],
                    buf, gsem)

        def wait_load(buf, gsem):
            pltpu.make_async_copy(v_lo.at[pl.ds(0, SROW)], buf, gsem).wait()

        def scatter_group(g, buf):
            for b in range(SK):
                idx = idx_v.at[g * SK + b]
                pltpu.sync_copy(buf.at[pl.ds(b * GCHUNK, GCHUNK)],
                                acc.at[idx], add=True)
                if with_cnt:
                    @pl.when(c == 1)
                    def _cnt():
                        pltpu.sync_copy(ones_v.at[pl.ds(0, GCHUNK)],
                                        cnt_acc.at[idx], add=True)

        fire_load(0, vbuf_a, gsem_a)
        fire_load(1, vbuf_b, gsem_b)

        def body(p, carry):
            g0 = 2 * p
            wait_load(vbuf_a, gsem_a)
            scatter_group(g0, vbuf_a)

            @pl.when(p < ngroups // 4 - 1)
            def _next_a():
                fire_load(g0 + 2, vbuf_a, gsem_a)

            wait_load(vbuf_b, gsem_b)
            scatter_group(g0 + 1, vbuf_b)

            @pl.when(p < ngroups // 4 - 1)
            def _next_b():
                fire_load(g0 + 3, vbuf_b, gsem_b)

            return carry

        lax.fori_loop(0, ngroups // 4, body, 0)

    zero_acc(acc, with_cnt=with_cnt)
    plsc.subcore_barrier()
    run_phase(acc, v_lo, v_hi, 0, with_cnt=with_cnt)
    run_phase(acc, v_lo, v_hi, 1, with_cnt=with_cnt)
    plsc.subcore_barrier()
    writeback(acc, sum_lo, sum_hi, with_cnt=with_cnt)

  return _sc_scatter_body


def _sc_scatter(quarters, col, n):
    E = quarters[0].shape[0]
    ept = E // NS
    npad = NS * 2 * SCH
    assert npad >= n
    mesh = plsc.VectorSubcoreMesh(core_axis_name="c", subcore_axis_name="s")
    shp = jax.ShapeDtypeStruct
    col4d = col.reshape(NS, 2, ept // 2 // GCHUNK, GCHUNK)
    z2 = jnp.zeros((WCH, QF), jnp.float32)
    z1 = jnp.zeros((WCH,), jnp.float32)
    on = jnp.ones((48,), jnp.float32)

    sums = []
    for k in range(4):
        with_cnt = False
        res = pl.kernel(
            _make_scatter_body(with_cnt),
            out_type=[shp((npad, QF), jnp.float32)] * 2
                     + [shp((npad,), jnp.float32)],
            mesh=mesh,
            compiler_params=pltpu.CompilerParams(use_tc_tiling_on_sc=False),
            scratch_types=[
                pltpu.VMEM((ept // 2 // GCHUNK, GCHUNK), jnp.int32),
                pltpu.VMEM((SROW, QF), jnp.float32),
                pltpu.VMEM((SROW, QF), jnp.float32),
                pltpu.VMEM((48,), jnp.float32),
                pltpu.VMEM((WCH, QF), jnp.float32),
                pltpu.VMEM((WCH,), jnp.float32),
                pltpu.VMEM_SHARED((npad, QF), jnp.float32),
                pltpu.VMEM_SHARED((npad,) if with_cnt else (8,), jnp.float32),
                pltpu.SemaphoreType.DMA,
                pltpu.SemaphoreType.DMA,
            ],
            name=f"sc_scatter_{k}",
        )(quarters[2 * k], quarters[2 * k + 1], col4d, z2, z1, on)
        sums.extend(r[:n] for r in res[:2])
    cnt0, cnt1 = _sc_count(col4d, z2, n)
    return sums + [cnt0, cnt1]


def _sc_count_body(col4d, zeros2d, ones2_h, cnt0, cnt1,
                   idx_v, ones_v, wbuf, acc):
    c = lax.axis_index("c")
    s = lax.axis_index("s")
    nchunks = idx_v.shape[0]
    node_base = s * 2 * SCH

    pltpu.sync_copy(col4d.at[s, c], idx_v)
    pltpu.sync_copy(ones2_h, ones_v)
    pltpu.sync_copy(zeros2d, wbuf)

    for k in range(17):
        pltpu.sync_copy(wbuf, acc.at[pl.ds(node_base + k * WCH, WCH)])
    plsc.subcore_barrier()

    def body(j, carry):
        pltpu.sync_copy(ones_v.at[pl.ds(0, GCHUNK)],
                        acc.at[idx_v.at[j]], add=True)
        return carry

    lax.fori_loop(0, nchunks, body, 0)
    plsc.subcore_barrier()

    for k in range(17):
        cb = node_base + k * WCH
        pltpu.sync_copy(acc.at[pl.ds(cb, WCH)], wbuf)

        @pl.when(c == 0)
        def _c0():
            pltpu.sync_copy(wbuf, cnt0.at[pl.ds(cb, WCH)])

        @pl.when(c == 1)
        def _c1():
            pltpu.sync_copy(wbuf, cnt1.at[pl.ds(cb, WCH)])


def _sc_count(col4d, z2, n):
    npad = NS * 2 * SCH
    nchunks = col4d.shape[2]
    mesh = plsc.VectorSubcoreMesh(core_axis_name="c", subcore_axis_name="s")
    shp = jax.ShapeDtypeStruct
    res = pl.kernel(
        _sc_count_body,
        out_type=[shp((npad, QF), jnp.float32)] * 2,
        mesh=mesh,
        compiler_params=pltpu.CompilerParams(use_tc_tiling_on_sc=False),
        scratch_types=[
            pltpu.VMEM((nchunks, GCHUNK), jnp.int32),
            pltpu.VMEM((48, QF), jnp.float32),
            pltpu.VMEM((WCH, QF), jnp.float32),
            pltpu.VMEM_SHARED((npad, QF), jnp.float32),
        ],
        name="sc_count",
    )(col4d, z2, jnp.ones((48, QF), jnp.float32))
    return res[0][:n, :1], res[1][:n, :1]


def _edge_stats_kernel(pag, ea, w1p, b1p, w1a, b1a,
                       sum_p, sq_p, sum_a, sq_a):
    @pl.when(pl.program_id(0) == 0)
    def _init():
        sum_p[...] = jnp.zeros_like(sum_p)
        sq_p[...] = jnp.zeros_like(sq_p)
        sum_a[...] = jnp.zeros_like(sum_a)
        sq_a[...] = jnp.zeros_like(sq_a)

    g = pag[...]
    e = ea[...]
    xp = jnp.concatenate([g[:, :F], e], axis=1)
    xa = jnp.concatenate([g[:, F:], e], axis=1)
    hp = jnp.maximum(jnp.dot(xp, w1p[...],
                             preferred_element_type=jnp.float32) + b1p[...], 0.0)
    ha = jnp.maximum(jnp.dot(xa, w1a[...],
                             preferred_element_type=jnp.float32) + b1a[...], 0.0)
    sum_p[...] += jnp.sum(hp, axis=0, keepdims=True)
    sq_p[...] += jnp.sum(hp * hp, axis=0, keepdims=True)
    sum_a[...] += jnp.sum(ha, axis=0, keepdims=True)
    sq_a[...] += jnp.sum(ha * ha, axis=0, keepdims=True)


def _edge_mlp_kernel(nrows, pag, ea,
                     w1p, b1p, gp, btp, w2p, b2p, w3p, b3p,
                     w1a, b1a, ga, bta, w2a, b2a, w3a, b3a,
                     sum_p, sq_p, sum_a, sq_a, *outs):
    g = pag[...]
    e = ea[...]

    def branch(xcols, w1, b1, gamma, beta, w2, b2, w3, b3, s, sq):
        x = jnp.concatenate([xcols, e], axis=1)
        h = jnp.maximum(jnp.dot(x, w1[...],
                                preferred_element_type=jnp.float32) + b1[...], 0.0)
        mu = s[...] / nrows
        var = sq[...] / nrows - mu * mu
        scale = gamma[...] * lax.rsqrt(var + 1e-5)
        hn = (h - mu) * scale + beta[...]
        h2 = jnp.maximum(jnp.dot(hn, w2[...],
                                 preferred_element_type=jnp.float32) + b2[...], 0.0)
        return jnp.dot(h2, w3[...], preferred_element_type=jnp.float32) + b3[...]

    op = branch(g[:, :F], w1p, b1p, gp, btp, w2p, b2p, w3p, b3p,
                sum_p, sq_p)
    oa = branch(g[:, F:], w1a, b1a, ga, bta, w2a, b2a, w3a, b3a,
                sum_a, sq_a)
    for q in range(4):
        outs[q][...] = op[:, q * QF:(q + 1) * QF]
        outs[4 + q][...] = oa[:, q * QF:(q + 1) * QF]


def _run_edge_mlps(pa_g, edge_attr, p1, a1):
    E = pa_g.shape[0]
    BE = 1600
    grid = (E // BE,)
    vec = lambda name, p: p[name].reshape(1, -1)
    full = lambda a: pl.BlockSpec(a.shape, lambda i: (0,) * a.ndim)
    blk = lambda w: pl.BlockSpec((BE, w), lambda i: (i, 0))
    stats_spec = [pl.BlockSpec((1, HID), lambda i: (0, 0))] * 4

    wargs1 = (p1['W1'], vec('b1', p1), a1['W1'], vec('b1', a1))
    stats = pl.pallas_call(
        _edge_stats_kernel,
        grid=grid,
        in_specs=[blk(2 * F), blk(F)] + [full(w) for w in wargs1],
        out_specs=stats_spec,
        out_shape=[jax.ShapeDtypeStruct((1, HID), jnp.float32)] * 4,
        compiler_params=pltpu.CompilerParams(
            dimension_semantics=("arbitrary",)),
    )(pa_g, edge_attr, *wargs1)

    def wset(p):
        return (p['W1'], vec('b1', p), vec('gamma', p), vec('beta', p),
                p['W2'], vec('b2', p), p['W3'], vec('b3', p))

    wargs2 = wset(p1) + wset(a1)
    outs = pl.pallas_call(
        functools.partial(_edge_mlp_kernel, float(E)),
        grid=grid,
        in_specs=([blk(2 * F), blk(F)] + [full(w) for w in wargs2]
                  + stats_spec),
        out_specs=[blk(QF)] * 8,
        out_shape=[jax.ShapeDtypeStruct((E, QF), jnp.float32)] * 8,
        compiler_params=pltpu.CompilerParams(
            dimension_semantics=("arbitrary",)),
    )(pa_g, edge_attr, *wargs2, *stats)
    return outs


def _node_x(nodes, sums, cnts):
    c = cnts[...]
    inv = 1.0 / jnp.maximum(c[:, :1] + c[:, 1:2], 1.0)
    return jnp.concatenate([nodes[...], sums[...] * inv], axis=1)


def _node_stats_kernel(nodes, sums, cnts, w1, b1, sum_o, sq_o):
    @pl.when(pl.program_id(0) == 0)
    def _init():
        sum_o[...] = jnp.zeros_like(sum_o)
        sq_o[...] = jnp.zeros_like(sq_o)

    x = _node_x(nodes, sums, cnts)
    h = jnp.maximum(jnp.dot(x, w1[...],
                            preferred_element_type=jnp.float32) + b1[...], 0.0)
    sum_o[...] += jnp.sum(h, axis=0, keepdims=True)
    sq_o[...] += jnp.sum(h * h, axis=0, keepdims=True)


def _node_mlp_kernel(nrows, nodes, sums, cnts,
                     w1, b1, gamma, beta, w2, b2, w3, b3, s, sq, out):
    x = _node_x(nodes, sums, cnts)
    h = jnp.maximum(jnp.dot(x, w1[...],
                            preferred_element_type=jnp.float32) + b1[...], 0.0)
    mu = s[...] / nrows
    var = sq[...] / nrows - mu * mu
    scale = gamma[...] * lax.rsqrt(var + 1e-5)
    hn = (h - mu) * scale + beta[...]
    h2 = jnp.maximum(jnp.dot(hn, w2[...],
                             preferred_element_type=jnp.float32) + b2[...], 0.0)
    out[...] = (nodes[...] + jnp.dot(h2, w3[...],
                                     preferred_element_type=jnp.float32)
                + b3[...])


def _run_node_mlp(nodes, sums, cnts, p):
    N = nodes.shape[0]
    BN = 4000
    grid = (N // BN,)
    vec = lambda name: p[name].reshape(1, -1)
    full = lambda a: pl.BlockSpec(a.shape, lambda i: (0,) * a.ndim)
    blk = lambda w: pl.BlockSpec((BN, w), lambda i: (i, 0))
    stats_spec = [pl.BlockSpec((1, HID), lambda i: (0, 0))] * 2
    data_specs = [blk(F), blk(F), blk(2)]

    wargs1 = (p['W1'], vec('b1'))
    stats = pl.pallas_call(
        _node_stats_kernel,
        grid=grid,
        in_specs=data_specs + [full(w) for w in wargs1],
        out_specs=stats_spec,
        out_shape=[jax.ShapeDtypeStruct((1, HID), jnp.float32)] * 2,
        compiler_params=pltpu.CompilerParams(
            dimension_semantics=("arbitrary",)),
    )(nodes, sums, cnts, *wargs1)

    wargs2 = (p['W1'], vec('b1'), vec('gamma'), vec('beta'),
              p['W2'], vec('b2'), p['W3'], vec('b3'))
    out = pl.pallas_call(
        functools.partial(_node_mlp_kernel, float(N)),
        grid=grid,
        in_specs=data_specs + [full(w) for w in wargs2] + stats_spec,
        out_specs=blk(F),
        out_shape=jax.ShapeDtypeStruct((N, F), jnp.float32),
        compiler_params=pltpu.CompilerParams(
            dimension_semantics=("arbitrary",)),
    )(nodes, sums, cnts, *wargs2, *stats)
    return out


def kernel(pos, ang, edge_index, edge_attr, params):
    row = edge_index[0]
    col = edge_index[1]
    n = pos.shape[0]

    pa = jnp.concatenate([pos, ang], axis=1)
    pa_g = _sc_gather(pa, row)

    equarters = _run_edge_mlps(pa_g, edge_attr, params['p1'], params['a1'])

    *sums, cnt0, cnt1 = _sc_scatter(equarters, col, n)
    cnts = jnp.concatenate([cnt0, cnt1], axis=1)
    sums_p = jnp.concatenate(sums[:4], axis=1)
    sums_a = jnp.concatenate(sums[4:], axis=1)

    u = _run_node_mlp(pos, sums_p, cnts, params['p2'])
    phi = _run_node_mlp(ang, sums_a, cnts, params['a2'])
    return (u, phi)
